# baseline (speedup 1.0000x reference)
import functools

import jax
import jax.numpy as jnp
from jax import lax
from jax.experimental import pallas as pl
from jax.experimental.pallas import tpu as pltpu
from jax.experimental.pallas import tpu_sc as plsc

NCORES = 2
NSUB = 16
NW = NCORES * NSUB
SUB = 80
KSUB = 25
GROUP_E = KSUB * SUB


def _edge_w_body(a0, a1, a2, a3, we, src, dst, w1_ref, w2_ref, sd_ref):
    c0, c1, c2, c3 = a0[...], a1[...], a2[...], a3[...]
    w = we[...]
    w1_ref[...] = c0 * w[0, 0] + c1 * w[0, 1] + c2 * w[0, 2] + c3 * w[0, 3]
    w2_ref[...] = c0 * w[1, 0] + c1 * w[1, 1] + c2 * w[1, 2] + c3 * w[1, 3]
    sd_ref[...] = src[...] * 16384 + dst[...]


def _edge_weights(ea0, ea1, ea2, ea3, westack, src, dst):
    E = ea0.shape[0]
    R, C = E // 128, 128
    sh = (R, C)
    col = pl.BlockSpec(sh, lambda: (0, 0))
    return pl.pallas_call(
        _edge_w_body,
        in_specs=[col, col, col, col, pl.BlockSpec((2, 4), lambda: (0, 0)),
                  col, col],
        out_specs=[col, col, col],
        out_shape=[jax.ShapeDtypeStruct(sh, jnp.float32)] * 2
        + [jax.ShapeDtypeStruct(sh, jnp.int32)],
    )(ea0.reshape(sh), ea1.reshape(sh), ea2.reshape(sh), ea3.reshape(sh),
      westack, src.reshape(sh), dst.reshape(sh))


def _make_sc_layer(N, D, E, compute_deg):
    EPW = E // NW
    KB = EPW // SUB
    assert KB * SUB == EPW and KB % 2 == 1 and KB >= 5
    ROWS_PER_SUB = N // NSUB
    NJ = D // 16

    out_type = [jax.ShapeDtypeStruct((NCORES, N, D), jnp.float32)]
    scratch = [
        pltpu.VMEM_SHARED((N, D), jnp.float32),
        pltpu.VMEM((SUB, D), jnp.float32),
        pltpu.VMEM((SUB, D), jnp.float32),
        pltpu.VMEM((KB, SUB), jnp.int32),
        pltpu.VMEM((KB, SUB), jnp.float32),
        pltpu.VMEM((SUB,), jnp.int32),
        pltpu.VMEM((SUB,), jnp.int32),
        pltpu.VMEM((SUB,), jnp.int32),
        pltpu.VMEM((SUB,), jnp.int32),
        pltpu.SemaphoreType.DMA,
        pltpu.SemaphoreType.DMA,
    ]
    if compute_deg:
        out_type.append(jax.ShapeDtypeStruct((NCORES, N), jnp.float32))
        scratch += [
            pltpu.VMEM_SHARED((N,), jnp.float32),
            pltpu.VMEM((SUB,), jnp.float32),
            pltpu.VMEM((400,), jnp.float32),
            pltpu.SemaphoreType.DMA,
        ]

    mesh = plsc.VectorSubcoreMesh(core_axis_name="c", subcore_axis_name="s")

    @functools.partial(pl.kernel, out_type=out_type, mesh=mesh,
                       scratch_types=scratch,
                       compiler_params=pltpu.CompilerParams(
                           use_tc_tiling_on_sc=False))
    def sc_layer(h_hbm, sd_hbm, w_hbm, *refs):
        if compute_deg:
            (acc_out, deg_out, acc_sh, rows_a, rows_b, sdg_v, wg_v,
             srcb_a, srcb_b, dstb_a, dstb_b, gsem, ssem,
             deg_sh, ones_v, dz_v, dsem) = refs
        else:
            (acc_out, acc_sh, rows_a, rows_b, sdg_v, wg_v,
             srcb_a, srcb_b, dstb_a, dstb_b, gsem, ssem) = refs
        cid = lax.axis_index("c")
        sid = lax.axis_index("s")
        wid = cid * NSUB + sid

        zeros16 = jnp.zeros((16,), jnp.float32)

        rows0 = wid * KB
        pltpu.sync_copy(sd_hbm.at[pl.ds(rows0, KB), :], sdg_v)
        pltpu.sync_copy(w_hbm.at[pl.ds(rows0, KB), :], wg_v)

        def _zrow(e, carry):
            for j in range(NJ):
                rows_a[e, pl.ds(j * 16, 16)] = zeros16
            return carry
        lax.fori_loop(0, SUB, _zrow, 0)

        r0 = sid * ROWS_PER_SUB
        full, rem = divmod(ROWS_PER_SUB, SUB)
        for t in range(full):
            pltpu.sync_copy(rows_a.at[pl.ds(0, SUB), :],
                            acc_sh.at[pl.ds(r0 + t * SUB, SUB), :])
        if rem:
            pltpu.sync_copy(rows_a.at[pl.ds(0, rem), :],
                            acc_sh.at[pl.ds(r0 + full * SUB, rem), :])

        if compute_deg:
            for t in range(0, SUB, 16):
                ones_v[pl.ds(t, 16)] = jnp.full((16,), 1.0, jnp.float32)
            for t in range(0, 400, 16):
                dz_v[pl.ds(t, 16)] = zeros16

            @pl.when(sid == 0)
            def _zero_deg():
                for t in range(N // 400):
                    pltpu.sync_copy(dz_v, deg_sh.at[pl.ds(t * 400, 400)])

        plsc.subcore_barrier()

        buf_a = (rows_a, srcb_a, dstb_a)
        buf_b = (rows_b, srcb_b, dstb_b)

        def unpack(k, bufs):
            _, srcb, dstb = bufs
            for t in range(SUB // 16):
                sl = pl.ds(t * 16, 16)
                v = sdg_v[k, sl]
                srcb[sl] = v >> 14
                dstb[sl] = v & 16383

        def g_start(bufs):
            rows, srcb, _ = bufs
            pltpu.async_copy(h_hbm.at[srcb], rows, gsem)

        def g_wait(bufs):
            rows, srcb, _ = bufs
            pltpu.make_async_copy(h_hbm.at[srcb], rows, gsem).wait()

        def s_start(bufs):
            rows, _, dstb = bufs
            pltpu.async_copy(rows, acc_sh.at[dstb], ssem, add=True)
            if compute_deg:
                pltpu.async_copy(ones_v, deg_sh.at[dstb], dsem, add=True)

        def s_wait(bufs):
            rows, _, dstb = bufs
            pltpu.make_async_copy(rows, acc_sh.at[dstb], ssem).wait()
            if compute_deg:
                pltpu.make_async_copy(ones_v, deg_sh.at[dstb], dsem).wait()

        def scale(k, bufs):
            rows = bufs[0]
            def _eb(eb, c2):
                e0 = eb * 16
                wv = wg_v[k, pl.ds(e0, 16)]
                for i in range(16):
                    ws = wv[i]
                    for j in range(NJ):
                        sl = pl.ds(j * 16, 16)
                        rows[e0 + i, sl] = rows[e0 + i, sl] * ws
                return c2
            lax.fori_loop(0, SUB // 16, _eb, 0)

        def phase(k, cur, nxt, wait_prev, prefetch):
            g_wait(cur)
            if wait_prev:
                s_wait(nxt)
            if prefetch:
                unpack(k + 1, nxt)
                g_start(nxt)
            scale(k, cur)
            s_start(cur)

        unpack(0, buf_a)
        g_start(buf_a)
        phase(jnp.int32(0), buf_a, buf_b, wait_prev=False, prefetch=True)

        def _pair(i, carry):
            phase(2 * i + 1, buf_b, buf_a, True, True)
            phase(2 * i + 2, buf_a, buf_b, True, True)
            return carry
        lax.fori_loop(0, (KB - 3) // 2, _pair, 0)

        phase(jnp.int32(KB - 2), buf_b, buf_a, True, True)
        phase(jnp.int32(KB - 1), buf_a, buf_b, True, False)
        s_wait(buf_a)

        plsc.subcore_barrier()

        pltpu.sync_copy(acc_sh.at[pl.ds(r0, ROWS_PER_SUB), :],
                        acc_out.at[cid, pl.ds(r0, ROWS_PER_SUB), :])
        if compute_deg:
            @pl.when(sid == 0)
            def _deg_out():
                pltpu.sync_copy(deg_sh, deg_out.at[cid])

    return sc_layer


def _update_body(acc0, acc1, deg0, deg1, h, wm, ws, b, out_ref):
    deg = deg0[...] + deg1[...] + 1.0
    agg = (acc0[...] + acc1[...]) * (1.0 / deg)
    y = jnp.dot(agg, wm[...], preferred_element_type=jnp.float32)
    y = y + jnp.dot(h[...], ws[...], preferred_element_type=jnp.float32)
    y = y + b[...]
    out_ref[...] = jnp.maximum(y, 0.0)


def _tc_update(acc0, acc1, deg0, deg1, h, Wm, Ws, b):
    N, D = h.shape
    R = 1000
    grid = N // R
    mat = pl.BlockSpec((R, D), lambda i: (i, 0))
    vec = pl.BlockSpec((R, 1), lambda i: (i, 0))
    wspec = pl.BlockSpec((D, D), lambda i: (0, 0))
    return pl.pallas_call(
        _update_body,
        grid=(grid,),
        in_specs=[mat, mat, vec, vec, mat, wspec, wspec,
                  pl.BlockSpec((D,), lambda i: (0,))],
        out_specs=mat,
        out_shape=jax.ShapeDtypeStruct((N, D), jnp.float32),
    )(acc0, acc1, deg0.reshape(N, 1), deg1.reshape(N, 1), h, Wm, Ws, b)


def kernel(x, edge_index, edge_attr, we1, Wm1, Ws1, b1, we2, Wm2, Ws2, b2):
    N, D = x.shape
    E = edge_index.shape[1]

    eaT = edge_attr.T
    w1, w2, sd = _edge_weights(eaT[0], eaT[1], eaT[2], eaT[3],
                               jnp.stack([we1, we2]),
                               edge_index[0], edge_index[1])

    sd2 = sd.reshape(E // SUB, SUB)
    w1_2 = w1.reshape(E // SUB, SUB)
    w2_2 = w2.reshape(E // SUB, SUB)

    sc_layer1 = _make_sc_layer(N, D, E, compute_deg=True)
    sc_layer2 = _make_sc_layer(N, D, E, compute_deg=False)

    acc, degp = sc_layer1(x, sd2, w1_2)
    h1 = _tc_update(acc[0], acc[1], degp[0], degp[1], x, Wm1, Ws1, b1)
    res2 = sc_layer2(h1, sd2, w2_2)
    acc2 = res2[0] if isinstance(res2, (list, tuple)) else res2
    h2 = _tc_update(acc2[0], acc2[1], degp[0], degp[1], h1, Wm2, Ws2, b2)
    return h2


# speedup vs baseline: 1.1688x; 1.1688x over previous
import functools

import jax
import jax.numpy as jnp
from jax import lax
from jax.experimental import pallas as pl
from jax.experimental.pallas import tpu as pltpu
from jax.experimental.pallas import tpu_sc as plsc

NCORES = 2
NSUB = 16
NW = NCORES * NSUB
SUB = 80
KSUB = 25
GROUP_E = KSUB * SUB


def _edge_w_body(a0, a1, a2, a3, we, src, dst, w1_ref, w2_ref, sd_ref):
    c0, c1, c2, c3 = a0[...], a1[...], a2[...], a3[...]
    w = we[...]
    w1_ref[...] = c0 * w[0, 0] + c1 * w[0, 1] + c2 * w[0, 2] + c3 * w[0, 3]
    w2_ref[...] = c0 * w[1, 0] + c1 * w[1, 1] + c2 * w[1, 2] + c3 * w[1, 3]
    sd_ref[...] = src[...] * 16384 + dst[...]


def _edge_weights(ea0, ea1, ea2, ea3, westack, src, dst):
    E = ea0.shape[0]
    R, C = E // 128, 128
    sh = (R, C)
    col = pl.BlockSpec(sh, lambda: (0, 0))
    return pl.pallas_call(
        _edge_w_body,
        in_specs=[col, col, col, col, pl.BlockSpec((2, 4), lambda: (0, 0)),
                  col, col],
        out_specs=[col, col, col],
        out_shape=[jax.ShapeDtypeStruct(sh, jnp.float32)] * 2
        + [jax.ShapeDtypeStruct(sh, jnp.int32)],
    )(ea0.reshape(sh), ea1.reshape(sh), ea2.reshape(sh), ea3.reshape(sh),
      westack, src.reshape(sh), dst.reshape(sh))


def _make_sc_layer(N, D, E, compute_deg):
    EPW = E // NW
    KB = EPW // SUB
    assert KB * SUB == EPW and KB % 2 == 1 and KB >= 5
    ROWS_PER_SUB = N // NSUB
    NJ = D // 16

    NBUF = 3
    PF = 2
    PRO = NBUF + KB % NBUF
    assert (KB - PRO - NBUF) % NBUF == 0 and KB >= PRO + 2 * NBUF
    CH = 25
    NCH = KB // CH
    WS = 3
    assert CH * NCH == KB and NCH >= WS and CH % NBUF != 0

    out_type = [jax.ShapeDtypeStruct((NCORES, N, D), jnp.float32)]
    scratch = [
        pltpu.VMEM_SHARED((N, D), jnp.float32),
        pltpu.VMEM((KB, SUB), jnp.int32),
        pltpu.VMEM((WS * CH, SUB), jnp.float32),
    ]
    for _ in range(NBUF):
        scratch += [
            pltpu.VMEM((SUB, D), jnp.float32),
            pltpu.VMEM((SUB,), jnp.int32),
            pltpu.VMEM((SUB,), jnp.int32),
        ]
    scratch += [
        pltpu.SemaphoreType.DMA,
        pltpu.SemaphoreType.DMA,
        pltpu.SemaphoreType.DMA,
    ]
    if compute_deg:
        out_type.append(jax.ShapeDtypeStruct((NCORES, N), jnp.float32))
        scratch += [
            pltpu.VMEM_SHARED((N,), jnp.float32),
            pltpu.VMEM((SUB,), jnp.float32),
            pltpu.VMEM((200,), jnp.float32),
            pltpu.SemaphoreType.DMA,
        ]

    mesh = plsc.VectorSubcoreMesh(core_axis_name="c", subcore_axis_name="s")

    @functools.partial(pl.kernel, out_type=out_type, mesh=mesh,
                       scratch_types=scratch,
                       compiler_params=pltpu.CompilerParams(
                           use_tc_tiling_on_sc=False))
    def sc_layer(h_hbm, sd_hbm, w_hbm, *refs):
        if compute_deg:
            acc_out, deg_out = refs[0], refs[1]
            refs = refs[2:]
        else:
            acc_out = refs[0]
            refs = refs[1:]
        acc_sh, sdg_v, wg_v = refs[0], refs[1], refs[2]
        bufs = [tuple(refs[3 + 3 * i:6 + 3 * i]) for i in range(NBUF)]
        gsem, ssem, wsem = (refs[3 + 3 * NBUF], refs[4 + 3 * NBUF],
                            refs[5 + 3 * NBUF])
        if compute_deg:
            deg_sh, ones_v, dz_v, dsem = refs[6 + 3 * NBUF:10 + 3 * NBUF]
        rows_a = bufs[0][0]
        cid = lax.axis_index("c")
        sid = lax.axis_index("s")
        wid = cid * NSUB + sid

        zeros16 = jnp.zeros((16,), jnp.float32)

        rows0 = wid * KB
        pltpu.sync_copy(sd_hbm.at[pl.ds(rows0, KB), :], sdg_v)

        def w_issue(c):
            pltpu.async_copy(
                w_hbm.at[pl.ds(rows0 + c * CH, CH), :],
                wg_v.at[pl.ds((c % WS) * CH, CH), :], wsem)

        def w_wait(c):
            pltpu.make_async_copy(
                w_hbm.at[pl.ds(rows0 + c * CH, CH), :],
                wg_v.at[pl.ds((c % WS) * CH, CH), :], wsem).wait()

        for c in range(WS):
            w_issue(c)
        w_wait(0)

        def _zrow(e, carry):
            for j in range(NJ):
                rows_a[e, pl.ds(j * 16, 16)] = zeros16
            return carry
        lax.fori_loop(0, SUB, _zrow, 0)

        r0 = sid * ROWS_PER_SUB
        full, rem = divmod(ROWS_PER_SUB, SUB)
        for t in range(full):
            pltpu.sync_copy(rows_a.at[pl.ds(0, SUB), :],
                            acc_sh.at[pl.ds(r0 + t * SUB, SUB), :])
        if rem:
            pltpu.sync_copy(rows_a.at[pl.ds(0, rem), :],
                            acc_sh.at[pl.ds(r0 + full * SUB, rem), :])

        if compute_deg:
            for t in range(0, SUB, 16):
                ones_v[pl.ds(t, 16)] = jnp.full((16,), 1.0, jnp.float32)
            for t in range(0, 192, 16):
                dz_v[pl.ds(t, 16)] = zeros16
            dz_v[pl.ds(184, 16)] = zeros16

            @pl.when(sid == 0)
            def _zero_deg():
                for t in range(N // 200):
                    pltpu.sync_copy(dz_v, deg_sh.at[pl.ds(t * 200, 200)])

        plsc.subcore_barrier()

        def unpack(k, b):
            _, srcb, dstb = b
            for t in range(SUB // 16):
                sl = pl.ds(t * 16, 16)
                v = sdg_v[k, sl]
                srcb[sl] = v >> 14
                dstb[sl] = v & 16383

        def g_start(b):
            rows, srcb, _ = b
            pltpu.async_copy(h_hbm.at[srcb], rows, gsem)

        def g_wait(b):
            rows, srcb, _ = b
            pltpu.make_async_copy(h_hbm.at[srcb], rows, gsem).wait()

        def s_start(b):
            rows, _, dstb = b
            pltpu.async_copy(rows, acc_sh.at[dstb], ssem, add=True)
            if compute_deg:
                pltpu.async_copy(ones_v, deg_sh.at[dstb], dsem, add=True)

        def s_wait(b):
            rows, _, dstb = b
            pltpu.make_async_copy(rows, acc_sh.at[dstb], ssem).wait()
            if compute_deg:
                pltpu.make_async_copy(ones_v, deg_sh.at[dstb], dsem).wait()

        def scale(k, b):
            rows = b[0]
            wrow = lax.rem(k, jnp.int32(WS * CH))
            def _eb(eb, c2):
                e0 = eb * 16
                wv = wg_v[wrow, pl.ds(e0, 16)]
                for i in range(16):
                    ws = wv[i]
                    for j in range(NJ):
                        sl = pl.ds(j * 16, 16)
                        rows[e0 + i, sl] = rows[e0 + i, sl] * ws
                return c2
            lax.fori_loop(0, SUB // 16, _eb, 0)

        def step(k, o, wait_prev, prefetch):
            cur = bufs[o]
            @pl.when(jnp.logical_and(lax.rem(k, jnp.int32(CH)) == 0, k > 0))
            def _wchunk():
                c = lax.div(k, jnp.int32(CH))
                w_wait(c)
                @pl.when(c + 2 < NCH)
                def _wfill():
                    w_issue(c + 2)
            g_wait(cur)
            nxt = bufs[(o + PF) % NBUF]
            if wait_prev:
                s_wait(nxt)
            if prefetch:
                unpack(k + PF, nxt)
                g_start(nxt)
            scale(k, cur)
            s_start(cur)

        for o in range(PF):
            unpack(o, bufs[o])
            g_start(bufs[o])
        for k in range(PRO):
            step(jnp.int32(k), k % NBUF, wait_prev=(k >= NBUF - PF),
                 prefetch=True)

        def _grp(g, carry):
            base = PRO + g * NBUF
            for o in range(NBUF):
                step(base + o, (PRO + o) % NBUF, True, True)
            return carry
        lax.fori_loop(0, (KB - PRO - NBUF) // NBUF, _grp, 0)

        for o in range(NBUF):
            k = KB - NBUF + o
            pf = k + PF < KB
            step(jnp.int32(k), k % NBUF, wait_prev=pf, prefetch=pf)
        for _ in range(NBUF):
            s_wait(bufs[0])

        plsc.subcore_barrier()

        pltpu.sync_copy(acc_sh.at[pl.ds(r0, ROWS_PER_SUB), :],
                        acc_out.at[cid, pl.ds(r0, ROWS_PER_SUB), :])
        if compute_deg:
            @pl.when(sid == 0)
            def _deg_out():
                pltpu.sync_copy(deg_sh, deg_out.at[cid])

    return sc_layer


def _update_body(acc0, acc1, deg0, deg1, h, wm, ws, b, out_ref):
    deg = deg0[...] + deg1[...] + 1.0
    agg = (acc0[...] + acc1[...]) * (1.0 / deg)
    y = jnp.dot(agg, wm[...], preferred_element_type=jnp.float32)
    y = y + jnp.dot(h[...], ws[...], preferred_element_type=jnp.float32)
    y = y + b[...]
    out_ref[...] = jnp.maximum(y, 0.0)


def _tc_update(acc0, acc1, deg0, deg1, h, Wm, Ws, b):
    N, D = h.shape
    R = 1000
    grid = N // R
    mat = pl.BlockSpec((R, D), lambda i: (i, 0))
    vec = pl.BlockSpec((R, 1), lambda i: (i, 0))
    wspec = pl.BlockSpec((D, D), lambda i: (0, 0))
    return pl.pallas_call(
        _update_body,
        grid=(grid,),
        in_specs=[mat, mat, vec, vec, mat, wspec, wspec,
                  pl.BlockSpec((D,), lambda i: (0,))],
        out_specs=mat,
        out_shape=jax.ShapeDtypeStruct((N, D), jnp.float32),
    )(acc0, acc1, deg0.reshape(N, 1), deg1.reshape(N, 1), h, Wm, Ws, b)


def kernel(x, edge_index, edge_attr, we1, Wm1, Ws1, b1, we2, Wm2, Ws2, b2):
    N, D = x.shape
    E = edge_index.shape[1]

    eaT = edge_attr.T
    w1, w2, sd = _edge_weights(eaT[0], eaT[1], eaT[2], eaT[3],
                               jnp.stack([we1, we2]),
                               edge_index[0], edge_index[1])

    sd2 = sd.reshape(E // SUB, SUB)
    w1_2 = w1.reshape(E // SUB, SUB)
    w2_2 = w2.reshape(E // SUB, SUB)

    sc_layer1 = _make_sc_layer(N, D, E, compute_deg=True)
    sc_layer2 = _make_sc_layer(N, D, E, compute_deg=False)

    acc, degp = sc_layer1(x, sd2, w1_2)
    h1 = _tc_update(acc[0], acc[1], degp[0], degp[1], x, Wm1, Ws1, b1)
    res2 = sc_layer2(h1, sd2, w2_2)
    acc2 = res2[0] if isinstance(res2, (list, tuple)) else res2
    h2 = _tc_update(acc2[0], acc2[1], degp[0], degp[1], h1, Wm2, Ws2, b2)
    return h2


# speedup vs baseline: 1.1834x; 1.0125x over previous
import functools

import jax
import jax.numpy as jnp
from jax import lax
from jax.experimental import pallas as pl
from jax.experimental.pallas import tpu as pltpu
from jax.experimental.pallas import tpu_sc as plsc

NCORES = 2
NSUB = 16
NW = NCORES * NSUB
SUB = 80
KSUB = 25
GROUP_E = KSUB * SUB


def _edge_w_body(a0, a1, a2, a3, we, src, dst, w1_ref, w2_ref, sd_ref):
    c0, c1, c2, c3 = a0[...], a1[...], a2[...], a3[...]
    w = we[...]
    w1_ref[...] = c0 * w[0, 0] + c1 * w[0, 1] + c2 * w[0, 2] + c3 * w[0, 3]
    w2_ref[...] = c0 * w[1, 0] + c1 * w[1, 1] + c2 * w[1, 2] + c3 * w[1, 3]
    sd_ref[...] = src[...] * 16384 + dst[...]


def _edge_weights(ea0, ea1, ea2, ea3, westack, src, dst):
    E = ea0.shape[0]
    R, C = E // 128, 128
    sh = (R, C)
    col = pl.BlockSpec(sh, lambda: (0, 0))
    return pl.pallas_call(
        _edge_w_body,
        in_specs=[col, col, col, col, pl.BlockSpec((2, 4), lambda: (0, 0)),
                  col, col],
        out_specs=[col, col, col],
        out_shape=[jax.ShapeDtypeStruct(sh, jnp.float32)] * 2
        + [jax.ShapeDtypeStruct(sh, jnp.int32)],
    )(ea0.reshape(sh), ea1.reshape(sh), ea2.reshape(sh), ea3.reshape(sh),
      westack, src.reshape(sh), dst.reshape(sh))


def _make_sc_layer(N, D, E, compute_deg):
    EPW = E // NW
    KB = EPW // SUB
    assert KB * SUB == EPW and KB % 2 == 1 and KB >= 5
    ROWS_PER_SUB = N // NSUB
    NJ = D // 16

    NBUF = 3
    PF = 2
    PRO = NBUF + KB % NBUF
    assert (KB - PRO - NBUF) % NBUF == 0 and KB >= PRO + 2 * NBUF
    CH = 25
    NCH = KB // CH
    WS = 3
    assert CH * NCH == KB and NCH >= WS

    out_type = [jax.ShapeDtypeStruct((NCORES, N, D), jnp.float32)]
    scratch = [
        pltpu.VMEM_SHARED((N, D), jnp.float32),
        pltpu.VMEM((KB, SUB), jnp.int32),
        pltpu.VMEM((WS * CH, SUB), jnp.float32),
    ]
    for _ in range(NBUF):
        scratch += [
            pltpu.VMEM((SUB, D), jnp.float32),
            pltpu.VMEM((SUB,), jnp.int32),
            pltpu.VMEM((SUB,), jnp.int32),
        ]
    scratch += [
        pltpu.SemaphoreType.DMA,
        pltpu.SemaphoreType.DMA,
        pltpu.SemaphoreType.DMA,
    ]
    if compute_deg:
        out_type.append(jax.ShapeDtypeStruct((NCORES, N), jnp.float32))
        scratch += [
            pltpu.VMEM_SHARED((N,), jnp.float32),
            pltpu.VMEM((SUB,), jnp.float32),
            pltpu.VMEM((200,), jnp.float32),
            pltpu.SemaphoreType.DMA,
        ]

    mesh = plsc.VectorSubcoreMesh(core_axis_name="c", subcore_axis_name="s")

    @functools.partial(pl.kernel, out_type=out_type, mesh=mesh,
                       scratch_types=scratch,
                       compiler_params=pltpu.CompilerParams(
                           use_tc_tiling_on_sc=False))
    def sc_layer(h_hbm, sd_hbm, w_hbm, *refs):
        if compute_deg:
            acc_out, deg_out = refs[0], refs[1]
            refs = refs[2:]
        else:
            acc_out = refs[0]
            refs = refs[1:]
        acc_sh, sdg_v, wg_v = refs[0], refs[1], refs[2]
        bufs = [tuple(refs[3 + 3 * i:6 + 3 * i]) for i in range(NBUF)]
        gsem, ssem, wsem = (refs[3 + 3 * NBUF], refs[4 + 3 * NBUF],
                            refs[5 + 3 * NBUF])
        if compute_deg:
            deg_sh, ones_v, dz_v, dsem = refs[6 + 3 * NBUF:10 + 3 * NBUF]
        rows_a = bufs[0][0]
        cid = lax.axis_index("c")
        sid = lax.axis_index("s")
        wid = cid * NSUB + sid

        zeros16 = jnp.zeros((16,), jnp.float32)

        rows0 = wid * KB
        pltpu.sync_copy(sd_hbm.at[pl.ds(rows0, KB), :], sdg_v)

        def w_issue(c):
            pltpu.async_copy(
                w_hbm.at[pl.ds(rows0 + c * CH, CH), :],
                wg_v.at[pl.ds((c % WS) * CH, CH), :], wsem)

        def w_wait(c):
            pltpu.make_async_copy(
                w_hbm.at[pl.ds(rows0 + c * CH, CH), :],
                wg_v.at[pl.ds((c % WS) * CH, CH), :], wsem).wait()

        for c in range(WS):
            w_issue(c)
        w_wait(0)

        def _zrow(e, carry):
            for j in range(NJ):
                rows_a[e, pl.ds(j * 16, 16)] = zeros16
            return carry
        lax.fori_loop(0, SUB, _zrow, 0)

        r0 = sid * ROWS_PER_SUB
        full, rem = divmod(ROWS_PER_SUB, SUB)
        for t in range(full):
            pltpu.sync_copy(rows_a.at[pl.ds(0, SUB), :],
                            acc_sh.at[pl.ds(r0 + t * SUB, SUB), :])
        if rem:
            pltpu.sync_copy(rows_a.at[pl.ds(0, rem), :],
                            acc_sh.at[pl.ds(r0 + full * SUB, rem), :])

        if compute_deg:
            for t in range(0, SUB, 16):
                ones_v[pl.ds(t, 16)] = jnp.full((16,), 1.0, jnp.float32)
            for t in range(0, 192, 16):
                dz_v[pl.ds(t, 16)] = zeros16
            dz_v[pl.ds(184, 16)] = zeros16

            @pl.when(sid == 0)
            def _zero_deg():
                for t in range(N // 200):
                    pltpu.sync_copy(dz_v, deg_sh.at[pl.ds(t * 200, 200)])

        plsc.subcore_barrier()

        def unpack(k, b):
            _, srcb, dstb = b
            for t in range(SUB // 16):
                sl = pl.ds(t * 16, 16)
                v = sdg_v[k, sl]
                srcb[sl] = v >> 14
                dstb[sl] = v & 16383

        def g_start(b):
            rows, srcb, _ = b
            pltpu.async_copy(h_hbm.at[srcb], rows, gsem)

        def g_wait(b):
            rows, srcb, _ = b
            pltpu.make_async_copy(h_hbm.at[srcb], rows, gsem).wait()

        def s_start(b):
            rows, _, dstb = b
            pltpu.async_copy(rows, acc_sh.at[dstb], ssem, add=True)
            if compute_deg:
                pltpu.async_copy(ones_v, deg_sh.at[dstb], dsem, add=True)

        def s_wait(b):
            rows, _, dstb = b
            pltpu.make_async_copy(rows, acc_sh.at[dstb], ssem).wait()
            if compute_deg:
                pltpu.make_async_copy(ones_v, deg_sh.at[dstb], dsem).wait()

        def scale(k, b):
            rows = b[0]
            wrow = lax.rem(k, jnp.int32(WS * CH))
            def _eb(eb, c2):
                e0 = eb * 16
                wv = wg_v[wrow, pl.ds(e0, 16)]
                for i in range(16):
                    ws = wv[i]
                    for j in range(NJ):
                        sl = pl.ds(j * 16, 16)
                        rows[e0 + i, sl] = rows[e0 + i, sl] * ws
                return c2
            lax.fori_loop(0, SUB // 16, _eb, 0)

        def step(k, o, wait_prev, prefetch):
            cur = bufs[o]
            @pl.when(jnp.logical_and(lax.rem(k, jnp.int32(CH)) == 0, k > 0))
            def _wchunk():
                c = lax.div(k, jnp.int32(CH))
                w_wait(c)
                @pl.when(c + 2 < NCH)
                def _wfill():
                    w_issue(c + 2)
            g_wait(cur)
            nxt = bufs[(o + PF) % NBUF]
            if wait_prev:
                s_wait(nxt)
            if prefetch:
                unpack(k + PF, nxt)
                g_start(nxt)
            scale(k, cur)
            s_start(cur)

        for o in range(PF):
            unpack(o, bufs[o])
            g_start(bufs[o])
        for k in range(PRO):
            step(jnp.int32(k), k % NBUF, wait_prev=(k >= NBUF - PF),
                 prefetch=True)

        def _grp(g, carry):
            base = PRO + g * NBUF
            for o in range(NBUF):
                step(base + o, (PRO + o) % NBUF, True, True)
            return carry
        lax.fori_loop(0, (KB - PRO - NBUF) // NBUF, _grp, 0)

        for o in range(NBUF):
            k = KB - NBUF + o
            pf = k + PF < KB
            step(jnp.int32(k), k % NBUF, wait_prev=pf, prefetch=pf)
        for _ in range(NBUF):
            s_wait(bufs[0])

        plsc.subcore_barrier()

        pltpu.sync_copy(acc_sh.at[pl.ds(r0, ROWS_PER_SUB), :],
                        acc_out.at[cid, pl.ds(r0, ROWS_PER_SUB), :])
        if compute_deg:
            @pl.when(sid == 0)
            def _deg_out():
                pltpu.sync_copy(deg_sh, deg_out.at[cid])

    return sc_layer


def _update_body(acc0, acc1, deg0, deg1, h, wm, ws, b, out_ref):
    deg = deg0[...] + deg1[...] + 1.0
    agg = (acc0[...] + acc1[...]) * (1.0 / deg)
    y = jnp.dot(agg, wm[...], preferred_element_type=jnp.float32)
    y = y + jnp.dot(h[...], ws[...], preferred_element_type=jnp.float32)
    y = y + b[...]
    out_ref[...] = jnp.maximum(y, 0.0)


def _tc_update(acc0, acc1, deg0, deg1, h, Wm, Ws, b):
    N, D = h.shape
    R = 2000
    grid = N // R
    mat = pl.BlockSpec((R, D), lambda i: (i, 0))
    vec = pl.BlockSpec((R, 1), lambda i: (i, 0))
    wspec = pl.BlockSpec((D, D), lambda i: (0, 0))
    return pl.pallas_call(
        _update_body,
        grid=(grid,),
        in_specs=[mat, mat, vec, vec, mat, wspec, wspec,
                  pl.BlockSpec((D,), lambda i: (0,))],
        out_specs=mat,
        out_shape=jax.ShapeDtypeStruct((N, D), jnp.float32),
    )(acc0, acc1, deg0.reshape(N, 1), deg1.reshape(N, 1), h, Wm, Ws, b)


def kernel(x, edge_index, edge_attr, we1, Wm1, Ws1, b1, we2, Wm2, Ws2, b2):
    N, D = x.shape
    E = edge_index.shape[1]

    eaT = edge_attr.T
    w1, w2, sd = _edge_weights(eaT[0], eaT[1], eaT[2], eaT[3],
                               jnp.stack([we1, we2]),
                               edge_index[0], edge_index[1])

    sd2 = sd.reshape(E // SUB, SUB)
    w1_2 = w1.reshape(E // SUB, SUB)
    w2_2 = w2.reshape(E // SUB, SUB)

    sc_layer1 = _make_sc_layer(N, D, E, compute_deg=True)
    sc_layer2 = _make_sc_layer(N, D, E, compute_deg=False)

    acc, degp = sc_layer1(x, sd2, w1_2)
    h1 = _tc_update(acc[0], acc[1], degp[0], degp[1], x, Wm1, Ws1, b1)
    res2 = sc_layer2(h1, sd2, w2_2)
    acc2 = res2[0] if isinstance(res2, (list, tuple)) else res2
    h2 = _tc_update(acc2[0], acc2[1], degp[0], degp[1], h1, Wm2, Ws2, b2)
    return h2
